# baseline (speedup 1.0000x reference)
import functools

import jax
import jax.numpy as jnp
from jax import lax
from jax.experimental import pallas as pl
from jax.experimental.pallas import tpu as pltpu
from jax.experimental.pallas import tpu_sc as plsc

NEG_SLOPE = 0.01
_CH = 128


def _leaky(v):
    return jnp.where(v >= 0, v, NEG_SLOPE * v)


def _node_body(x_ref, wst_ref, bs_ref, wtt_ref, bt_ref, a1st_ref, a1tt_ref,
               xs_ref, xt_ref, ps_ref, pt_ref):
    xb = x_ref[...]
    xs = jnp.dot(xb, wst_ref[...], preferred_element_type=jnp.float32) + bs_ref[...]
    xt = jnp.dot(xb, wtt_ref[...], preferred_element_type=jnp.float32) + bt_ref[...]
    xs_ref[...] = xs
    xt_ref[...] = xt
    ps_ref[...] = jnp.dot(xs, a1st_ref[...], preferred_element_type=jnp.float32)
    pt_ref[...] = jnp.dot(xt, a1tt_ref[...], preferred_element_type=jnp.float32)


def _node_transform(x, wst, bs2, wtt, bt2, a1st, a1tt):
    n, din = x.shape
    dout = wst.shape[1]
    bn = 1000 if n % 1000 == 0 else n
    grid = (n // bn,)
    full = lambda shape: pl.BlockSpec(shape, lambda i: (0, 0))
    blkn = pl.BlockSpec((bn, din), lambda i: (i, 0))
    blko = pl.BlockSpec((bn, dout), lambda i: (i, 0))
    outs = [jax.ShapeDtypeStruct((n, dout), jnp.float32)] * 4
    return pl.pallas_call(
        _node_body,
        grid=grid,
        in_specs=[blkn, full((din, dout)), full((1, dout)), full((din, dout)),
                  full((1, dout)), full((dout, dout)), full((dout, dout))],
        out_specs=[blko, blko, blko, blko],
        out_shape=outs,
    )(x, wst, bs2, wtt, bt2, a1st, a1tt)


_NB = 4


def _pad8(x):
    return -(-x // 8) * 8


def _make_gather_sum(e, dout, nc, ns):
    n_chunks = e // _CH
    nw = nc * ns
    per_w = _pad8(-(-n_chunks // nw))
    n_iter = -(-(per_w + 2) // _NB) * _NB
    mesh = plsc.VectorSubcoreMesh(core_axis_name="c", subcore_axis_name="s")

    @functools.partial(
        pl.kernel,
        out_type=jax.ShapeDtypeStruct((e, dout), jnp.float32),
        mesh=mesh,
        scratch_types=[
            pltpu.VMEM((per_w, _CH), jnp.int32),
            pltpu.VMEM((per_w, _CH), jnp.int32),
            pltpu.VMEM((_NB, _CH, dout), jnp.float32),
        ] + [pltpu.SemaphoreType.DMA] * (2 * _NB),
    )
    def gather_sum(ps_hbm, pt_hbm, src_hbm, tgt_hbm, out_hbm,
                   idxs, idxt, buf, *sems):
        sem_a = sems[:_NB]
        sem_w = sems[_NB:]
        wid = lax.axis_index("s") * nc + lax.axis_index("c")
        c0 = wid * per_w
        m = jnp.minimum(per_w, n_chunks - c0)
        pltpu.sync_copy(src_hbm.at[pl.ds(c0, per_w)], idxs)
        pltpu.sync_copy(tgt_hbm.at[pl.ds(c0, per_w)], idxt)

        def wait64(sem):
            pltpu.make_async_copy(ps_hbm.at[pl.ds(0, _CH)], buf.at[0], sem).wait()

        def fire_a(c, b):
            pltpu.async_copy(ps_hbm.at[idxs.at[c]], buf.at[b], sem_a[b])

        fire_a(0, 0)
        fire_a(1, 1)

        def step(u, carry):
            for r in range(_NB):
                t = u * _NB + r
                b_cur = r
                b_nxt = (r + 2) % _NB
                b_wb = (r + 3) % _NB

                @pl.when((t >= 2) & (t <= m + 1))
                def _():
                    wait64(sem_w[b_nxt])

                @pl.when(t + 2 < m)
                def _():
                    fire_a(t + 2, b_nxt)

                @pl.when(t < m)
                def _():
                    wait64(sem_a[b_cur])
                    pltpu.async_copy(pt_hbm.at[idxt.at[t]], buf.at[b_cur],
                                     sem_a[b_cur], add=True)

                @pl.when((t >= 1) & (t <= m))
                def _():
                    wait64(sem_a[b_wb])
                    base = (c0 + t - 1) * _CH
                    pltpu.async_copy(buf.at[b_wb],
                                     out_hbm.at[pl.ds(base, _CH)], sem_w[b_wb])
            return carry

        lax.fori_loop(0, n_iter // _NB, step, 0)

    return gather_sum


def _edge_mlp_body(s_ref, ea_ref, wet_ref, be_ref, a1et_ref, b1_ref,
                   a2t_ref, b2_ref, a3_ref, b3_ref, att_ref):
    me = jnp.dot(wet_ref[...], a1et_ref[...], preferred_element_type=jnp.float32)
    ce = jnp.dot(be_ref[...], a1et_ref[...], preferred_element_type=jnp.float32) + b1_ref[...]
    h1 = _leaky(s_ref[...] + jnp.dot(ea_ref[...], me, preferred_element_type=jnp.float32) + ce)
    h2 = _leaky(jnp.dot(h1, a2t_ref[...], preferred_element_type=jnp.float32) + b2_ref[...])
    e = jnp.sum(h2 * a3_ref[...], axis=1, keepdims=True) + b3_ref[...]
    att_ref[...] = jnp.maximum(e, 0.0)


def _edge_mlp(s, ea, wet, be2, a1et, b12, a2t, b22, a32, b32):
    e, dout = s.shape
    de = ea.shape[1]
    dh = a2t.shape[1]
    be_blk = 2560 if e % 2560 == 0 else _CH
    grid = (e // be_blk,)
    full = lambda shape: pl.BlockSpec(shape, lambda i: tuple(0 for _ in shape))
    return pl.pallas_call(
        _edge_mlp_body,
        grid=grid,
        in_specs=[pl.BlockSpec((be_blk, dout), lambda i: (i, 0)),
                  pl.BlockSpec((be_blk, de), lambda i: (i, 0)),
                  full((de, dout)), full((1, dout)), full((dout, dout)),
                  full((1, dout)), full((dout, dh)), full((1, dh)),
                  full((1, dh)), full((1, 1))],
        out_specs=pl.BlockSpec((be_blk, 1), lambda i: (i, 0)),
        out_shape=jax.ShapeDtypeStruct((e, 1), jnp.float32),
    )(s, ea, wet, be2, a1et, b12, a2t, b22, a32, b32)


def _make_scatter(e, n, dout, nc, ns):
    n_chunks = e // _CH
    nw = nc * ns
    zrows = 80
    n_row_chunks = n // zrows
    mesh = plsc.VectorSubcoreMesh(core_axis_name="c", subcore_axis_name="s")

    per_w = _pad8(-(-n_chunks // nw))
    nb4 = 2

    @functools.partial(
        pl.kernel,
        out_type=jax.ShapeDtypeStruct((nc, n, dout), jnp.float32),
        mesh=mesh,
        scratch_types=[
            pltpu.VMEM_SHARED((n, dout), jnp.float32),
            pltpu.VMEM((nb4, 2, _CH), jnp.int32),
            pltpu.VMEM((nb4, _CH), jnp.float32),
            pltpu.VMEM((nb4, _CH, dout), jnp.float32),
            pltpu.VMEM((zrows, dout), jnp.float32),
        ] + [pltpu.SemaphoreType.DMA] * (2 * nb4),
    )
    def scatter(xs_hbm, meta_hbm, att_hbm, out_hbm,
                acc, metab, attb, buf, zbuf, *sems):
        sem_a = sems[:nb4]
        sem_s = sems[nb4:]
        cid = lax.axis_index("c")
        sid = lax.axis_index("s")
        wid = sid * nc + cid
        c0 = wid * per_w
        m = jnp.minimum(per_w, n_chunks - c0)

        def zrow(j, c2):
            for k in range(dout // 16):
                zbuf[j, pl.ds(k * 16, 16)] = jnp.zeros((16,), jnp.float32)
            return c2

        lax.fori_loop(0, zrows, zrow, 0)
        nzmine = (n_row_chunks - sid + ns - 1) // ns

        def zchunk(i, carry):
            rbase = (sid + i * ns) * zrows
            pltpu.sync_copy(zbuf, acc.at[pl.ds(rbase, zrows)])
            return carry

        lax.fori_loop(0, nzmine, zchunk, 0)
        plsc.subcore_barrier()

        def wait64(sem):
            pltpu.make_async_copy(xs_hbm.at[pl.ds(0, _CH)], buf.at[0], sem).wait()

        def fire_a(c, b):
            pltpu.sync_copy(meta_hbm.at[c0 + c], metab.at[b])
            pltpu.sync_copy(att_hbm.at[c0 + c], attb.at[b])
            pltpu.async_copy(xs_hbm.at[metab.at[b, 0]], buf.at[b], sem_a[b])

        def scale_rows(b):
            def grp(g, c2):
                av = attb[b, pl.ds(g * 16, 16)]
                for r in range(16):
                    ab = jnp.full((16,), av[r], dtype=jnp.float32)
                    j = g * 16 + r
                    for k in range(dout // 16):
                        sl = pl.ds(k * 16, 16)
                        buf[b, j, sl] = buf[b, j, sl] * ab
                return c2

            lax.fori_loop(0, _CH // 16, grp, 0)

        fire_a(0, 0)

        def step(u, carry):
            for r in range(nb4):
                t = u * nb4 + r
                b_cur = r
                b_nxt = (r + 1) % nb4

                @pl.when((t >= 1) & (t <= m))
                def _():
                    wait64(sem_s[b_nxt])

                @pl.when(t + 1 < m)
                def _():
                    fire_a(t + 1, b_nxt)

                @pl.when(t < m)
                def _():
                    wait64(sem_a[b_cur])
                    scale_rows(b_cur)
                    pltpu.async_copy(buf.at[b_cur], acc.at[metab.at[b_cur, 1]],
                                     sem_s[b_cur], add=True)
            return carry

        lax.fori_loop(0, (per_w + 2) // nb4, step, 0)
        plsc.subcore_barrier()

        def ochunk(i, carry):
            rbase = (sid + i * ns) * zrows
            pltpu.sync_copy(acc.at[pl.ds(rbase, zrows)],
                            out_hbm.at[cid, pl.ds(rbase, zrows)])
            return carry

        lax.fori_loop(0, nzmine, ochunk, 0)

    return scatter


def _combine_body(p0_ref, p1_ref, xt_ref, out_ref):
    out_ref[...] = p0_ref[...] + p1_ref[...] + xt_ref[...]


def _combine(p0, p1, xt):
    n, dout = xt.shape
    bn = 1000 if n % 1000 == 0 else n
    blk = pl.BlockSpec((bn, dout), lambda i: (i, 0))
    return pl.pallas_call(
        _combine_body,
        grid=(n // bn,),
        in_specs=[blk, blk, blk],
        out_specs=blk,
        out_shape=jax.ShapeDtypeStruct((n, dout), jnp.float32),
    )(p0, p1, xt)


def kernel(x, edge_index, edge_attr, Ws, bs, Wt, bt, We, be, A1, b1, A2, b2, A3, b3):
    n, din = x.shape
    e = edge_index.shape[1]
    dout = Ws.shape[0]
    dh = A2.shape[0]

    info = plsc.get_sparse_core_info()
    nc, ns = info.num_cores, info.num_subcores
    n_chunks = e // _CH

    src = edge_index[0].astype(jnp.int32).reshape(n_chunks, _CH)
    tgt = edge_index[1].astype(jnp.int32).reshape(n_chunks, _CH)

    a1t = A1.T
    a1st, a1tt, a1et = a1t[:dout], a1t[dout:2 * dout], a1t[2 * dout:]

    xs, xt, ps, pt = _node_transform(
        x, Ws.T, bs.reshape(1, dout), Wt.T, bt.reshape(1, dout), a1st, a1tt)

    n_half = 2
    eh = e // n_half
    ch_h = eh // _CH
    per_w = _pad8(-(-ch_h // (nc * ns)))
    pad_rows = nc * ns * per_w - ch_h

    gather_half = _make_gather_sum(eh, dout, nc, ns)
    atts = []
    for h in range(n_half):
        rows = slice(h * ch_h, (h + 1) * ch_h)
        src_p = jnp.pad(src[rows], ((0, pad_rows), (0, 0)))
        tgt_p = jnp.pad(tgt[rows], ((0, pad_rows), (0, 0)))
        s_sum = gather_half(ps, pt, src_p, tgt_p)
        atts.append(_edge_mlp(
            s_sum, edge_attr[h * eh:(h + 1) * eh], We.T, be.reshape(1, dout),
            a1et, b1.reshape(1, dout), A2.T, b2.reshape(1, dh),
            A3.reshape(1, dh), b3.reshape(1, 1)))

    att = jnp.concatenate(atts, axis=0)
    meta = jnp.stack([src, tgt], axis=1)

    parts = _make_scatter(e, n, dout, nc, ns)(
        xs, meta, att.reshape(n_chunks, _CH))

    return _combine(parts[0], parts[1], xt)


# speedup vs baseline: 3.1178x; 3.1178x over previous
import functools

import jax
import jax.numpy as jnp
from jax import lax
from jax.experimental import pallas as pl
from jax.experimental.pallas import tpu as pltpu
from jax.experimental.pallas import tpu_sc as plsc

NEG_SLOPE = 0.01
_CH = 128


def _leaky(v):
    return jnp.where(v >= 0, v, NEG_SLOPE * v)


def _node_body(x_ref, wst_ref, bs_ref, wtt_ref, bt_ref, a1st_ref, a1tt_ref,
               xs_ref, xt_ref, ps_ref, pt_ref):
    xb = x_ref[...]
    xs = jnp.dot(xb, wst_ref[...], preferred_element_type=jnp.float32) + bs_ref[...]
    xt = jnp.dot(xb, wtt_ref[...], preferred_element_type=jnp.float32) + bt_ref[...]
    xs_ref[...] = xs
    xt_ref[...] = xt
    ps_ref[...] = jnp.dot(xs, a1st_ref[...], preferred_element_type=jnp.float32)
    pt_ref[...] = jnp.dot(xt, a1tt_ref[...], preferred_element_type=jnp.float32)


def _node_transform(x, wst, bs2, wtt, bt2, a1st, a1tt):
    n, din = x.shape
    dout = wst.shape[1]
    bn = 1000 if n % 1000 == 0 else n
    grid = (n // bn,)
    full = lambda shape: pl.BlockSpec(shape, lambda i: (0, 0))
    blkn = pl.BlockSpec((bn, din), lambda i: (i, 0))
    blko = pl.BlockSpec((bn, dout), lambda i: (i, 0))
    outs = [jax.ShapeDtypeStruct((n, dout), jnp.float32)] * 4
    return pl.pallas_call(
        _node_body,
        grid=grid,
        in_specs=[blkn, full((din, dout)), full((1, dout)), full((din, dout)),
                  full((1, dout)), full((dout, dout)), full((dout, dout))],
        out_specs=[blko, blko, blko, blko],
        out_shape=outs,
    )(x, wst, bs2, wtt, bt2, a1st, a1tt)


_NB = 4


def _pad8(x):
    return -(-x // 8) * 8


def _make_gather_sum(e, dout, nc, ns):
    n_chunks = e // _CH
    nw = nc * ns
    per_w = _pad8(-(-n_chunks // nw))
    n_iter = -(-(per_w + 2) // _NB) * _NB
    mesh = plsc.VectorSubcoreMesh(core_axis_name="c", subcore_axis_name="s")

    @functools.partial(
        pl.kernel,
        out_type=jax.ShapeDtypeStruct((e, dout), jnp.float32),
        mesh=mesh,
        scratch_types=[
            pltpu.VMEM((per_w, _CH), jnp.int32),
            pltpu.VMEM((per_w, _CH), jnp.int32),
            pltpu.VMEM((_NB, _CH, dout), jnp.float32),
        ] + [pltpu.SemaphoreType.DMA] * (2 * _NB),
    )
    def gather_sum(ps_hbm, pt_hbm, src_hbm, tgt_hbm, out_hbm,
                   idxs, idxt, buf, *sems):
        sem_a = sems[:_NB]
        sem_w = sems[_NB:]
        wid = lax.axis_index("s") * nc + lax.axis_index("c")
        c0 = wid * per_w
        m = jnp.minimum(per_w, n_chunks - c0)
        pltpu.sync_copy(src_hbm.at[pl.ds(c0, per_w)], idxs)
        pltpu.sync_copy(tgt_hbm.at[pl.ds(c0, per_w)], idxt)

        def wait64(sem):
            pltpu.make_async_copy(ps_hbm.at[pl.ds(0, _CH)], buf.at[0], sem).wait()

        def fire_a(c, b):
            pltpu.async_copy(ps_hbm.at[idxs.at[c]], buf.at[b], sem_a[b])

        fire_a(0, 0)
        fire_a(1, 1)

        def step(u, carry):
            for r in range(_NB):
                t = u * _NB + r
                b_cur = r
                b_nxt = (r + 2) % _NB
                b_wb = (r + 3) % _NB

                @pl.when((t >= 2) & (t <= m + 1))
                def _():
                    wait64(sem_w[b_nxt])

                @pl.when(t + 2 < m)
                def _():
                    fire_a(t + 2, b_nxt)

                @pl.when(t < m)
                def _():
                    wait64(sem_a[b_cur])
                    pltpu.async_copy(pt_hbm.at[idxt.at[t]], buf.at[b_cur],
                                     sem_a[b_cur], add=True)

                @pl.when((t >= 1) & (t <= m))
                def _():
                    wait64(sem_a[b_wb])
                    base = (c0 + t - 1) * _CH
                    pltpu.async_copy(buf.at[b_wb],
                                     out_hbm.at[pl.ds(base, _CH)], sem_w[b_wb])
            return carry

        lax.fori_loop(0, n_iter // _NB, step, 0)

    return gather_sum


def _edge_mlp_body(s_ref, ea_ref, wet_ref, be_ref, a1et_ref, b1_ref,
                   a2t_ref, b2_ref, a3_ref, b3_ref, att_ref):
    me = jnp.dot(wet_ref[...], a1et_ref[...], preferred_element_type=jnp.float32)
    ce = jnp.dot(be_ref[...], a1et_ref[...], preferred_element_type=jnp.float32) + b1_ref[...]
    h1 = _leaky(s_ref[...] + jnp.dot(ea_ref[...], me, preferred_element_type=jnp.float32) + ce)
    h2 = _leaky(jnp.dot(h1, a2t_ref[...], preferred_element_type=jnp.float32) + b2_ref[...])
    e = jnp.sum(h2 * a3_ref[...], axis=1, keepdims=True) + b3_ref[...]
    att_ref[...] = jnp.maximum(e, 0.0)


def _edge_mlp(s, ea, wet, be2, a1et, b12, a2t, b22, a32, b32):
    e, dout = s.shape
    de = ea.shape[1]
    dh = a2t.shape[1]
    be_blk = 2560 if e % 2560 == 0 else _CH
    grid = (e // be_blk,)
    full = lambda shape: pl.BlockSpec(shape, lambda i: tuple(0 for _ in shape))
    return pl.pallas_call(
        _edge_mlp_body,
        grid=grid,
        in_specs=[pl.BlockSpec((be_blk, dout), lambda i: (i, 0)),
                  pl.BlockSpec((be_blk, de), lambda i: (i, 0)),
                  full((de, dout)), full((1, dout)), full((dout, dout)),
                  full((1, dout)), full((dout, dh)), full((1, dh)),
                  full((1, dh)), full((1, 1))],
        out_specs=pl.BlockSpec((be_blk, 1), lambda i: (i, 0)),
        out_shape=jax.ShapeDtypeStruct((e, 1), jnp.float32),
    )(s, ea, wet, be2, a1et, b12, a2t, b22, a32, b32)


def _make_scatter(e, n, dout, nc, ns):
    n_chunks = e // _CH
    nw = nc * ns
    zrows = 80
    n_row_chunks = n // zrows
    mesh = plsc.VectorSubcoreMesh(core_axis_name="c", subcore_axis_name="s")

    per_w = _pad8(-(-n_chunks // nw))
    nb4 = 2

    @functools.partial(
        pl.kernel,
        out_type=jax.ShapeDtypeStruct((nc, n, dout), jnp.float32),
        mesh=mesh,
        scratch_types=[
            pltpu.VMEM_SHARED((n, dout), jnp.float32),
            pltpu.VMEM((nb4, 2, _CH), jnp.int32),
            pltpu.VMEM((nb4, _CH), jnp.float32),
            pltpu.VMEM((nb4, _CH, dout), jnp.float32),
            pltpu.VMEM((zrows, dout), jnp.float32),
        ] + [pltpu.SemaphoreType.DMA] * (2 * nb4),
    )
    def scatter(xs_hbm, meta_hbm, att_hbm, out_hbm,
                acc, metab, attb, buf, zbuf, *sems):
        sem_a = sems[:nb4]
        sem_s = sems[nb4:]
        cid = lax.axis_index("c")
        sid = lax.axis_index("s")
        wid = sid * nc + cid
        c0 = wid * per_w
        m = jnp.minimum(per_w, n_chunks - c0)

        def zrow(j, c2):
            for k in range(dout // 16):
                zbuf[j, pl.ds(k * 16, 16)] = jnp.zeros((16,), jnp.float32)
            return c2

        lax.fori_loop(0, zrows, zrow, 0)
        nzmine = (n_row_chunks - sid + ns - 1) // ns

        def zchunk(i, carry):
            rbase = (sid + i * ns) * zrows
            pltpu.sync_copy(zbuf, acc.at[pl.ds(rbase, zrows)])
            return carry

        lax.fori_loop(0, nzmine, zchunk, 0)
        plsc.subcore_barrier()

        def wait64(sem):
            pltpu.make_async_copy(xs_hbm.at[pl.ds(0, _CH)], buf.at[0], sem).wait()

        def fire_a(c, b):
            pltpu.sync_copy(meta_hbm.at[c0 + c], metab.at[b])
            pltpu.sync_copy(att_hbm.at[c0 + c], attb.at[b])
            pltpu.async_copy(xs_hbm.at[metab.at[b, 0]], buf.at[b], sem_a[b])

        def scale_rows(b):
            def grp(g, c2):
                av = attb[b, pl.ds(g * 16, 16)]
                for r in range(16):
                    ab = jnp.full((16,), av[r], dtype=jnp.float32)
                    j = g * 16 + r
                    for k in range(dout // 16):
                        sl = pl.ds(k * 16, 16)
                        buf[b, j, sl] = buf[b, j, sl] * ab
                return c2

            lax.fori_loop(0, _CH // 16, grp, 0)

        fire_a(0, 0)

        def step(u, carry):
            for r in range(nb4):
                t = u * nb4 + r
                b_cur = r
                b_nxt = (r + 1) % nb4

                @pl.when((t >= 1) & (t <= m))
                def _():
                    wait64(sem_s[b_nxt])

                @pl.when(t + 1 < m)
                def _():
                    fire_a(t + 1, b_nxt)

                @pl.when(t < m)
                def _():
                    wait64(sem_a[b_cur])
                    scale_rows(b_cur)
                    pltpu.async_copy(buf.at[b_cur], acc.at[metab.at[b_cur, 1]],
                                     sem_s[b_cur], add=True)
            return carry

        lax.fori_loop(0, (per_w + 2) // nb4, step, 0)
        plsc.subcore_barrier()

        def ochunk(i, carry):
            rbase = (sid + i * ns) * zrows
            pltpu.sync_copy(acc.at[pl.ds(rbase, zrows)],
                            out_hbm.at[cid, pl.ds(rbase, zrows)])
            return carry

        lax.fori_loop(0, nzmine, ochunk, 0)

    return scatter


def _combine_body(p0_ref, p1_ref, xt_ref, out_ref):
    out_ref[...] = p0_ref[...] + p1_ref[...] + xt_ref[...]


def _combine(p0, p1, xt):
    n, dout = xt.shape
    bn = 1000 if n % 1000 == 0 else n
    blk = pl.BlockSpec((bn, dout), lambda i: (i, 0))
    return pl.pallas_call(
        _combine_body,
        grid=(n // bn,),
        in_specs=[blk, blk, blk],
        out_specs=blk,
        out_shape=jax.ShapeDtypeStruct((n, dout), jnp.float32),
    )(p0, p1, xt)


def kernel(x, edge_index, edge_attr, Ws, bs, Wt, bt, We, be, A1, b1, A2, b2, A3, b3):
    n, din = x.shape
    e = edge_index.shape[1]
    dout = Ws.shape[0]
    dh = A2.shape[0]

    info = plsc.get_sparse_core_info()
    nc, ns = info.num_cores, info.num_subcores
    n_chunks = e // _CH

    src = edge_index[0].astype(jnp.int32).reshape(n_chunks, _CH)
    tgt = edge_index[1].astype(jnp.int32).reshape(n_chunks, _CH)

    a1t = A1.T
    a1st, a1tt, a1et = a1t[:dout], a1t[dout:2 * dout], a1t[2 * dout:]

    xs, xt, ps, pt = _node_transform(
        x, Ws.T, bs.reshape(1, dout), Wt.T, bt.reshape(1, dout), a1st, a1tt)

    per_w = _pad8(-(-n_chunks // (nc * ns)))
    pad_rows = nc * ns * per_w - n_chunks
    src_p = jnp.pad(src, ((0, pad_rows), (0, 0)))
    tgt_p = jnp.pad(tgt, ((0, pad_rows), (0, 0)))

    s_sum = _make_gather_sum(e, dout, nc, ns)(ps, pt, src_p, tgt_p)
    att = _edge_mlp(s_sum, edge_attr, We.T, be.reshape(1, dout), a1et,
                    b1.reshape(1, dout), A2.T, b2.reshape(1, dh),
                    A3.reshape(1, dh), b3.reshape(1, 1))

    meta = jnp.stack([src, tgt], axis=1)

    parts = _make_scatter(e, n, dout, nc, ns)(
        xs, meta, att.reshape(n_chunks, _CH))

    return _combine(parts[0], parts[1], xt)


# speedup vs baseline: 3.3359x; 1.0700x over previous
import functools

import jax
import jax.numpy as jnp
from jax import lax
from jax.experimental import pallas as pl
from jax.experimental.pallas import tpu as pltpu
from jax.experimental.pallas import tpu_sc as plsc

NEG_SLOPE = 0.01
_CH = 128


def _leaky(v):
    return jnp.where(v >= 0, v, NEG_SLOPE * v)


def _node_body(x_ref, wst_ref, bs_ref, wtt_ref, bt_ref, a1st_ref, a1tt_ref,
               xs_ref, xt_ref, ps_ref, pt_ref):
    xb = x_ref[...]
    xs = jnp.dot(xb, wst_ref[...], preferred_element_type=jnp.float32) + bs_ref[...]
    xt = jnp.dot(xb, wtt_ref[...], preferred_element_type=jnp.float32) + bt_ref[...]
    xs_ref[...] = xs
    xt_ref[...] = xt
    ps_ref[...] = jnp.dot(xs, a1st_ref[...], preferred_element_type=jnp.float32)
    pt_ref[...] = jnp.dot(xt, a1tt_ref[...], preferred_element_type=jnp.float32)


def _node_transform(x, wst, bs2, wtt, bt2, a1st, a1tt):
    n, din = x.shape
    dout = wst.shape[1]
    bn = 1000 if n % 1000 == 0 else n
    grid = (n // bn,)
    full = lambda shape: pl.BlockSpec(shape, lambda i: (0, 0))
    blkn = pl.BlockSpec((bn, din), lambda i: (i, 0))
    blko = pl.BlockSpec((bn, dout), lambda i: (i, 0))
    outs = [jax.ShapeDtypeStruct((n, dout), jnp.float32)] * 4
    return pl.pallas_call(
        _node_body,
        grid=grid,
        in_specs=[blkn, full((din, dout)), full((1, dout)), full((din, dout)),
                  full((1, dout)), full((dout, dout)), full((dout, dout))],
        out_specs=[blko, blko, blko, blko],
        out_shape=outs,
    )(x, wst, bs2, wtt, bt2, a1st, a1tt)


_NB = 4


def _pad8(x):
    return -(-x // 8) * 8


def _make_gather_sum(e, dout, nc, ns):
    n_chunks = e // _CH
    nw = nc * ns
    per_w = _pad8(-(-n_chunks // nw))
    n_iter = -(-(per_w + 2) // _NB) * _NB
    mesh = plsc.VectorSubcoreMesh(core_axis_name="c", subcore_axis_name="s")

    @functools.partial(
        pl.kernel,
        out_type=jax.ShapeDtypeStruct((e, dout), jnp.float32),
        mesh=mesh,
        scratch_types=[
            pltpu.VMEM((per_w, _CH), jnp.int32),
            pltpu.VMEM((per_w, _CH), jnp.int32),
            pltpu.VMEM((_NB, _CH, dout), jnp.float32),
        ] + [pltpu.SemaphoreType.DMA] * (2 * _NB),
    )
    def gather_sum(ps_hbm, pt_hbm, src_hbm, tgt_hbm, out_hbm,
                   idxs, idxt, buf, *sems):
        sem_a = sems[:_NB]
        sem_w = sems[_NB:]
        wid = lax.axis_index("s") * nc + lax.axis_index("c")
        c0 = wid * per_w
        m = jnp.minimum(per_w, n_chunks - c0)
        pltpu.sync_copy(src_hbm.at[pl.ds(c0, per_w)], idxs)
        pltpu.sync_copy(tgt_hbm.at[pl.ds(c0, per_w)], idxt)

        def wait64(sem):
            pltpu.make_async_copy(ps_hbm.at[pl.ds(0, _CH)], buf.at[0], sem).wait()

        def fire_a(c, b):
            pltpu.async_copy(ps_hbm.at[idxs.at[c]], buf.at[b], sem_a[b])

        fire_a(0, 0)
        fire_a(1, 1)

        def step(u, carry):
            for r in range(_NB):
                t = u * _NB + r
                b_cur = r
                b_nxt = (r + 2) % _NB
                b_wb = (r + 3) % _NB

                @pl.when((t >= 2) & (t <= m + 1))
                def _():
                    wait64(sem_w[b_nxt])

                @pl.when(t + 2 < m)
                def _():
                    fire_a(t + 2, b_nxt)

                @pl.when(t < m)
                def _():
                    wait64(sem_a[b_cur])
                    pltpu.async_copy(pt_hbm.at[idxt.at[t]], buf.at[b_cur],
                                     sem_a[b_cur], add=True)

                @pl.when((t >= 1) & (t <= m))
                def _():
                    wait64(sem_a[b_wb])
                    base = (c0 + t - 1) * _CH
                    pltpu.async_copy(buf.at[b_wb],
                                     out_hbm.at[pl.ds(base, _CH)], sem_w[b_wb])
            return carry

        lax.fori_loop(0, n_iter // _NB, step, 0)

    return gather_sum


def _edge_mlp_body(s_ref, ea_ref, wet_ref, be_ref, a1et_ref, b1_ref,
                   a2t_ref, b2_ref, a3_ref, b3_ref, att_ref):
    me = jnp.dot(wet_ref[...], a1et_ref[...], preferred_element_type=jnp.float32)
    ce = jnp.dot(be_ref[...], a1et_ref[...], preferred_element_type=jnp.float32) + b1_ref[...]
    h1 = _leaky(s_ref[...] + jnp.dot(ea_ref[...], me, preferred_element_type=jnp.float32) + ce)
    h2 = _leaky(jnp.dot(h1, a2t_ref[...], preferred_element_type=jnp.float32) + b2_ref[...])
    e = jnp.sum(h2 * a3_ref[...], axis=1, keepdims=True) + b3_ref[...]
    att_ref[...] = jnp.maximum(e, 0.0)


def _edge_mlp(s, ea, wet, be2, a1et, b12, a2t, b22, a32, b32):
    e, dout = s.shape
    de = ea.shape[1]
    dh = a2t.shape[1]
    be_blk = 6400 if e % 6400 == 0 else _CH
    grid = (e // be_blk,)
    full = lambda shape: pl.BlockSpec(shape, lambda i: tuple(0 for _ in shape))
    return pl.pallas_call(
        _edge_mlp_body,
        grid=grid,
        in_specs=[pl.BlockSpec((be_blk, dout), lambda i: (i, 0)),
                  pl.BlockSpec((be_blk, de), lambda i: (i, 0)),
                  full((de, dout)), full((1, dout)), full((dout, dout)),
                  full((1, dout)), full((dout, dh)), full((1, dh)),
                  full((1, dh)), full((1, 1))],
        out_specs=pl.BlockSpec((be_blk, 1), lambda i: (i, 0)),
        out_shape=jax.ShapeDtypeStruct((e, 1), jnp.float32),
    )(s, ea, wet, be2, a1et, b12, a2t, b22, a32, b32)


def _make_scatter(e, n, dout, nc, ns):
    n_chunks = e // _CH
    nw = nc * ns
    zrows = 80
    n_row_chunks = n // zrows
    mesh = plsc.VectorSubcoreMesh(core_axis_name="c", subcore_axis_name="s")

    per_w = _pad8(-(-n_chunks // nw))
    nb4 = 2

    @functools.partial(
        pl.kernel,
        out_type=jax.ShapeDtypeStruct((nc, n, dout), jnp.float32),
        mesh=mesh,
        scratch_types=[
            pltpu.VMEM_SHARED((n, dout), jnp.float32),
            pltpu.VMEM((nb4, 2, _CH), jnp.int32),
            pltpu.VMEM((nb4, _CH), jnp.float32),
            pltpu.VMEM((nb4, _CH, dout), jnp.float32),
            pltpu.VMEM((zrows, dout), jnp.float32),
        ] + [pltpu.SemaphoreType.DMA] * (2 * nb4),
    )
    def scatter(xs_hbm, meta_hbm, att_hbm, out_hbm,
                acc, metab, attb, buf, zbuf, *sems):
        sem_a = sems[:nb4]
        sem_s = sems[nb4:]
        cid = lax.axis_index("c")
        sid = lax.axis_index("s")
        wid = sid * nc + cid
        c0 = wid * per_w
        m = jnp.minimum(per_w, n_chunks - c0)

        def zrow(j, c2):
            for k in range(dout // 16):
                zbuf[j, pl.ds(k * 16, 16)] = jnp.zeros((16,), jnp.float32)
            return c2

        lax.fori_loop(0, zrows, zrow, 0)
        nzmine = (n_row_chunks - sid + ns - 1) // ns

        def zchunk(i, carry):
            rbase = (sid + i * ns) * zrows
            pltpu.sync_copy(zbuf, acc.at[pl.ds(rbase, zrows)])
            return carry

        lax.fori_loop(0, nzmine, zchunk, 0)
        plsc.subcore_barrier()

        def wait64(sem):
            pltpu.make_async_copy(xs_hbm.at[pl.ds(0, _CH)], buf.at[0], sem).wait()

        def fire_a(c, b):
            pltpu.sync_copy(meta_hbm.at[c0 + c], metab.at[b])
            pltpu.sync_copy(att_hbm.at[c0 + c], attb.at[b])
            pltpu.async_copy(xs_hbm.at[metab.at[b, 0]], buf.at[b], sem_a[b])

        def scale_rows(b):
            def grp(g, c2):
                av = attb[b, pl.ds(g * 16, 16)]
                for r in range(16):
                    ab = jnp.full((16,), av[r], dtype=jnp.float32)
                    j = g * 16 + r
                    for k in range(dout // 16):
                        sl = pl.ds(k * 16, 16)
                        buf[b, j, sl] = buf[b, j, sl] * ab
                return c2

            lax.fori_loop(0, _CH // 16, grp, 0)

        fire_a(0, 0)

        def step(u, carry):
            for r in range(nb4):
                t = u * nb4 + r
                b_cur = r
                b_nxt = (r + 1) % nb4

                @pl.when((t >= 1) & (t <= m))
                def _():
                    wait64(sem_s[b_nxt])

                @pl.when(t + 1 < m)
                def _():
                    fire_a(t + 1, b_nxt)

                @pl.when(t < m)
                def _():
                    wait64(sem_a[b_cur])
                    scale_rows(b_cur)
                    pltpu.async_copy(buf.at[b_cur], acc.at[metab.at[b_cur, 1]],
                                     sem_s[b_cur], add=True)
            return carry

        lax.fori_loop(0, (per_w + 2) // nb4, step, 0)
        plsc.subcore_barrier()

        def ochunk(i, carry):
            rbase = (sid + i * ns) * zrows
            pltpu.sync_copy(acc.at[pl.ds(rbase, zrows)],
                            out_hbm.at[cid, pl.ds(rbase, zrows)])
            return carry

        lax.fori_loop(0, nzmine, ochunk, 0)

    return scatter


def _combine_body(p0_ref, p1_ref, xt_ref, out_ref):
    out_ref[...] = p0_ref[...] + p1_ref[...] + xt_ref[...]


def _combine(p0, p1, xt):
    n, dout = xt.shape
    bn = 1000 if n % 1000 == 0 else n
    blk = pl.BlockSpec((bn, dout), lambda i: (i, 0))
    return pl.pallas_call(
        _combine_body,
        grid=(n // bn,),
        in_specs=[blk, blk, blk],
        out_specs=blk,
        out_shape=jax.ShapeDtypeStruct((n, dout), jnp.float32),
    )(p0, p1, xt)


def kernel(x, edge_index, edge_attr, Ws, bs, Wt, bt, We, be, A1, b1, A2, b2, A3, b3):
    n, din = x.shape
    e = edge_index.shape[1]
    dout = Ws.shape[0]
    dh = A2.shape[0]

    info = plsc.get_sparse_core_info()
    nc, ns = info.num_cores, info.num_subcores
    n_chunks = e // _CH

    src = edge_index[0].astype(jnp.int32).reshape(n_chunks, _CH)
    tgt = edge_index[1].astype(jnp.int32).reshape(n_chunks, _CH)

    a1t = A1.T
    a1st, a1tt, a1et = a1t[:dout], a1t[dout:2 * dout], a1t[2 * dout:]

    xs, xt, ps, pt = _node_transform(
        x, Ws.T, bs.reshape(1, dout), Wt.T, bt.reshape(1, dout), a1st, a1tt)

    per_w = _pad8(-(-n_chunks // (nc * ns)))
    pad_rows = nc * ns * per_w - n_chunks
    src_p = jnp.pad(src, ((0, pad_rows), (0, 0)))
    tgt_p = jnp.pad(tgt, ((0, pad_rows), (0, 0)))

    s_sum = _make_gather_sum(e, dout, nc, ns)(ps, pt, src_p, tgt_p)
    att = _edge_mlp(s_sum, edge_attr, We.T, be.reshape(1, dout), a1et,
                    b1.reshape(1, dout), A2.T, b2.reshape(1, dh),
                    A3.reshape(1, dh), b3.reshape(1, 1))

    meta = jnp.stack([src, tgt], axis=1)

    parts = _make_scatter(e, n, dout, nc, ns)(
        xs, meta, att.reshape(n_chunks, _CH))

    return _combine(parts[0], parts[1], xt)


# speedup vs baseline: 3.3744x; 1.0115x over previous
import functools

import jax
import jax.numpy as jnp
from jax import lax
from jax.experimental import pallas as pl
from jax.experimental.pallas import tpu as pltpu
from jax.experimental.pallas import tpu_sc as plsc

NEG_SLOPE = 0.01
_CH = 128


def _leaky(v):
    return jnp.where(v >= 0, v, NEG_SLOPE * v)


def _node_body(x_ref, wst_ref, bs_ref, wtt_ref, bt_ref, a1st_ref, a1tt_ref,
               xs_ref, xt_ref, ps_ref, pt_ref):
    xb = x_ref[...]
    xs = jnp.dot(xb, wst_ref[...], preferred_element_type=jnp.float32) + bs_ref[...]
    xt = jnp.dot(xb, wtt_ref[...], preferred_element_type=jnp.float32) + bt_ref[...]
    xs_ref[...] = xs
    xt_ref[...] = xt
    ps_ref[...] = jnp.dot(xs, a1st_ref[...], preferred_element_type=jnp.float32)
    pt_ref[...] = jnp.dot(xt, a1tt_ref[...], preferred_element_type=jnp.float32)


def _node_transform(x, wst, bs2, wtt, bt2, a1st, a1tt):
    n, din = x.shape
    dout = wst.shape[1]
    bn = 2000 if n % 2000 == 0 else n
    grid = (n // bn,)
    full = lambda shape: pl.BlockSpec(shape, lambda i: (0, 0))
    blkn = pl.BlockSpec((bn, din), lambda i: (i, 0))
    blko = pl.BlockSpec((bn, dout), lambda i: (i, 0))
    outs = [jax.ShapeDtypeStruct((n, dout), jnp.float32)] * 4
    return pl.pallas_call(
        _node_body,
        grid=grid,
        in_specs=[blkn, full((din, dout)), full((1, dout)), full((din, dout)),
                  full((1, dout)), full((dout, dout)), full((dout, dout))],
        out_specs=[blko, blko, blko, blko],
        out_shape=outs,
    )(x, wst, bs2, wtt, bt2, a1st, a1tt)


_NB = 4


def _pad8(x):
    return -(-x // 8) * 8


def _make_gather_sum(e, dout, nc, ns):
    n_chunks = e // _CH
    nw = nc * ns
    per_w = _pad8(-(-n_chunks // nw))
    n_iter = -(-(per_w + 2) // _NB) * _NB
    mesh = plsc.VectorSubcoreMesh(core_axis_name="c", subcore_axis_name="s")

    @functools.partial(
        pl.kernel,
        out_type=jax.ShapeDtypeStruct((e, dout), jnp.float32),
        mesh=mesh,
        scratch_types=[
            pltpu.VMEM((per_w, _CH), jnp.int32),
            pltpu.VMEM((per_w, _CH), jnp.int32),
            pltpu.VMEM((_NB, _CH, dout), jnp.float32),
        ] + [pltpu.SemaphoreType.DMA] * (2 * _NB),
    )
    def gather_sum(ps_hbm, pt_hbm, src_hbm, tgt_hbm, out_hbm,
                   idxs, idxt, buf, *sems):
        sem_a = sems[:_NB]
        sem_w = sems[_NB:]
        wid = lax.axis_index("s") * nc + lax.axis_index("c")
        c0 = wid * per_w
        m = jnp.minimum(per_w, n_chunks - c0)
        pltpu.sync_copy(src_hbm.at[pl.ds(c0, per_w)], idxs)
        pltpu.sync_copy(tgt_hbm.at[pl.ds(c0, per_w)], idxt)

        def wait64(sem):
            pltpu.make_async_copy(ps_hbm.at[pl.ds(0, _CH)], buf.at[0], sem).wait()

        def fire_a(c, b):
            pltpu.async_copy(ps_hbm.at[idxs.at[c]], buf.at[b], sem_a[b])

        fire_a(0, 0)
        fire_a(1, 1)

        def step(u, carry):
            for r in range(_NB):
                t = u * _NB + r
                b_cur = r
                b_nxt = (r + 2) % _NB
                b_wb = (r + 3) % _NB

                @pl.when((t >= 2) & (t <= m + 1))
                def _():
                    wait64(sem_w[b_nxt])

                @pl.when(t + 2 < m)
                def _():
                    fire_a(t + 2, b_nxt)

                @pl.when(t < m)
                def _():
                    wait64(sem_a[b_cur])
                    pltpu.async_copy(pt_hbm.at[idxt.at[t]], buf.at[b_cur],
                                     sem_a[b_cur], add=True)

                @pl.when((t >= 1) & (t <= m))
                def _():
                    wait64(sem_a[b_wb])
                    base = (c0 + t - 1) * _CH
                    pltpu.async_copy(buf.at[b_wb],
                                     out_hbm.at[pl.ds(base, _CH)], sem_w[b_wb])
            return carry

        lax.fori_loop(0, n_iter // _NB, step, 0)

    return gather_sum


def _edge_mlp_body(s_ref, ea_ref, wet_ref, be_ref, a1et_ref, b1_ref,
                   a2t_ref, b2_ref, a3_ref, b3_ref, att_ref):
    me = jnp.dot(wet_ref[...], a1et_ref[...], preferred_element_type=jnp.float32)
    ce = jnp.dot(be_ref[...], a1et_ref[...], preferred_element_type=jnp.float32) + b1_ref[...]
    h1 = _leaky(s_ref[...] + jnp.dot(ea_ref[...], me, preferred_element_type=jnp.float32) + ce)
    h2 = _leaky(jnp.dot(h1, a2t_ref[...], preferred_element_type=jnp.float32) + b2_ref[...])
    e = jnp.sum(h2 * a3_ref[...], axis=1, keepdims=True) + b3_ref[...]
    att_ref[...] = jnp.maximum(e, 0.0)


def _edge_mlp(s, ea, wet, be2, a1et, b12, a2t, b22, a32, b32):
    e, dout = s.shape
    de = ea.shape[1]
    dh = a2t.shape[1]
    be_blk = 12800 if e % 12800 == 0 else _CH
    grid = (e // be_blk,)
    full = lambda shape: pl.BlockSpec(shape, lambda i: tuple(0 for _ in shape))
    return pl.pallas_call(
        _edge_mlp_body,
        grid=grid,
        in_specs=[pl.BlockSpec((be_blk, dout), lambda i: (i, 0)),
                  pl.BlockSpec((be_blk, de), lambda i: (i, 0)),
                  full((de, dout)), full((1, dout)), full((dout, dout)),
                  full((1, dout)), full((dout, dh)), full((1, dh)),
                  full((1, dh)), full((1, 1))],
        out_specs=pl.BlockSpec((be_blk, 1), lambda i: (i, 0)),
        out_shape=jax.ShapeDtypeStruct((e, 1), jnp.float32),
    )(s, ea, wet, be2, a1et, b12, a2t, b22, a32, b32)


def _make_scatter(e, n, dout, nc, ns):
    n_chunks = e // _CH
    nw = nc * ns
    zrows = 80
    n_row_chunks = n // zrows
    mesh = plsc.VectorSubcoreMesh(core_axis_name="c", subcore_axis_name="s")

    per_w = _pad8(-(-n_chunks // nw))
    nb4 = 2

    @functools.partial(
        pl.kernel,
        out_type=jax.ShapeDtypeStruct((nc, n, dout), jnp.float32),
        mesh=mesh,
        scratch_types=[
            pltpu.VMEM_SHARED((n, dout), jnp.float32),
            pltpu.VMEM((nb4, 2, _CH), jnp.int32),
            pltpu.VMEM((nb4, _CH), jnp.float32),
            pltpu.VMEM((nb4, _CH, dout), jnp.float32),
            pltpu.VMEM((zrows, dout), jnp.float32),
        ] + [pltpu.SemaphoreType.DMA] * (2 * nb4),
    )
    def scatter(xs_hbm, meta_hbm, att_hbm, out_hbm,
                acc, metab, attb, buf, zbuf, *sems):
        sem_a = sems[:nb4]
        sem_s = sems[nb4:]
        cid = lax.axis_index("c")
        sid = lax.axis_index("s")
        wid = sid * nc + cid
        c0 = wid * per_w
        m = jnp.minimum(per_w, n_chunks - c0)

        def zrow(j, c2):
            for k in range(dout // 16):
                zbuf[j, pl.ds(k * 16, 16)] = jnp.zeros((16,), jnp.float32)
            return c2

        lax.fori_loop(0, zrows, zrow, 0)
        nzmine = (n_row_chunks - sid + ns - 1) // ns

        def zchunk(i, carry):
            rbase = (sid + i * ns) * zrows
            pltpu.sync_copy(zbuf, acc.at[pl.ds(rbase, zrows)])
            return carry

        lax.fori_loop(0, nzmine, zchunk, 0)
        plsc.subcore_barrier()

        def wait64(sem):
            pltpu.make_async_copy(xs_hbm.at[pl.ds(0, _CH)], buf.at[0], sem).wait()

        def fire_a(c, b):
            pltpu.sync_copy(meta_hbm.at[c0 + c], metab.at[b])
            pltpu.sync_copy(att_hbm.at[c0 + c], attb.at[b])
            pltpu.async_copy(xs_hbm.at[metab.at[b, 0]], buf.at[b], sem_a[b])

        def scale_rows(b):
            def grp(g, c2):
                av = attb[b, pl.ds(g * 16, 16)]
                for r in range(16):
                    ab = jnp.full((16,), av[r], dtype=jnp.float32)
                    j = g * 16 + r
                    for k in range(dout // 16):
                        sl = pl.ds(k * 16, 16)
                        buf[b, j, sl] = buf[b, j, sl] * ab
                return c2

            lax.fori_loop(0, _CH // 16, grp, 0)

        fire_a(0, 0)

        def step(u, carry):
            for r in range(nb4):
                t = u * nb4 + r
                b_cur = r
                b_nxt = (r + 1) % nb4

                @pl.when((t >= 1) & (t <= m))
                def _():
                    wait64(sem_s[b_nxt])

                @pl.when(t + 1 < m)
                def _():
                    fire_a(t + 1, b_nxt)

                @pl.when(t < m)
                def _():
                    wait64(sem_a[b_cur])
                    scale_rows(b_cur)
                    pltpu.async_copy(buf.at[b_cur], acc.at[metab.at[b_cur, 1]],
                                     sem_s[b_cur], add=True)
            return carry

        lax.fori_loop(0, (per_w + 2) // nb4, step, 0)
        plsc.subcore_barrier()

        def ochunk(i, carry):
            rbase = (sid + i * ns) * zrows
            pltpu.sync_copy(acc.at[pl.ds(rbase, zrows)],
                            out_hbm.at[cid, pl.ds(rbase, zrows)])
            return carry

        lax.fori_loop(0, nzmine, ochunk, 0)

    return scatter


def _combine_body(p0_ref, p1_ref, xt_ref, out_ref):
    out_ref[...] = p0_ref[...] + p1_ref[...] + xt_ref[...]


def _combine(p0, p1, xt):
    n, dout = xt.shape
    bn = 2000 if n % 2000 == 0 else n
    blk = pl.BlockSpec((bn, dout), lambda i: (i, 0))
    return pl.pallas_call(
        _combine_body,
        grid=(n // bn,),
        in_specs=[blk, blk, blk],
        out_specs=blk,
        out_shape=jax.ShapeDtypeStruct((n, dout), jnp.float32),
    )(p0, p1, xt)


def kernel(x, edge_index, edge_attr, Ws, bs, Wt, bt, We, be, A1, b1, A2, b2, A3, b3):
    n, din = x.shape
    e = edge_index.shape[1]
    dout = Ws.shape[0]
    dh = A2.shape[0]

    info = plsc.get_sparse_core_info()
    nc, ns = info.num_cores, info.num_subcores
    n_chunks = e // _CH

    src = edge_index[0].astype(jnp.int32).reshape(n_chunks, _CH)
    tgt = edge_index[1].astype(jnp.int32).reshape(n_chunks, _CH)

    a1t = A1.T
    a1st, a1tt, a1et = a1t[:dout], a1t[dout:2 * dout], a1t[2 * dout:]

    xs, xt, ps, pt = _node_transform(
        x, Ws.T, bs.reshape(1, dout), Wt.T, bt.reshape(1, dout), a1st, a1tt)

    per_w = _pad8(-(-n_chunks // (nc * ns)))
    pad_rows = nc * ns * per_w - n_chunks
    src_p = jnp.pad(src, ((0, pad_rows), (0, 0)))
    tgt_p = jnp.pad(tgt, ((0, pad_rows), (0, 0)))

    s_sum = _make_gather_sum(e, dout, nc, ns)(ps, pt, src_p, tgt_p)
    att = _edge_mlp(s_sum, edge_attr, We.T, be.reshape(1, dout), a1et,
                    b1.reshape(1, dout), A2.T, b2.reshape(1, dh),
                    A3.reshape(1, dh), b3.reshape(1, 1))

    meta = jnp.stack([src, tgt], axis=1)

    parts = _make_scatter(e, n, dout, nc, ns)(
        xs, meta, att.reshape(n_chunks, _CH))

    return _combine(parts[0], parts[1], xt)
